# baseline (speedup 1.0000x reference)
import functools

import jax
import jax.numpy as jnp
from jax import lax
from jax.experimental import pallas as pl
from jax.experimental.pallas import tpu as pltpu
from jax.experimental.pallas import tpu_sc as plsc

_B, _L, _NUM_PIDS = 128, 4096, 256
_EPS = 1e-06
_LANES = 16
_NC, _NS = 2, 16
_NW = _NC * _NS
_ROWS_PER_W = _B // _NW
_SLAB = _ROWS_PER_W * _L
_CHUNKS = _SLAB // _LANES
_TAB = _ROWS_PER_W * _NUM_PIDS
_TAB_CHUNKS = _TAB // _LANES


def _sc_partials(yhat, y, pm):
    mesh = plsc.VectorSubcoreMesh(core_axis_name="c", subcore_axis_name="s")

    @functools.partial(
        pl.kernel,
        mesh=mesh,
        out_type=jax.ShapeDtypeStruct((_NC, _LANES), jnp.float32),
        compiler_params=pltpu.CompilerParams(needs_layout_passes=False),
        scratch_types=[
            pltpu.VMEM((_SLAB,), jnp.float32),
            pltpu.VMEM((_SLAB,), jnp.float32),
            pltpu.VMEM((_SLAB,), jnp.int32),
            pltpu.VMEM((_TAB,), jnp.float32),
            pltpu.VMEM((_TAB,), jnp.float32),
            pltpu.VMEM((_LANES,), jnp.float32),
            pltpu.VMEM((_LANES,), jnp.float32),
            pltpu.VMEM((_NS * _LANES,), jnp.float32),
            pltpu.VMEM((_NS * _LANES,), jnp.float32),
            pltpu.VMEM_SHARED((_NS * _LANES,), jnp.float32),
            pltpu.VMEM_SHARED((_NS * _LANES,), jnp.float32),
            pltpu.SemaphoreType.DMA,
        ],
    )
    def k(yhat_hbm, y_hbm, pm_hbm, out_hbm,
          yh_v, yy_v, pm_v, pred_t, true_t, st_sq, st_cnt,
          all_sq, all_cnt, sh_sq, sh_cnt, sem):
        cid = lax.axis_index("c")
        sid = lax.axis_index("s")
        wid = cid * _NS + sid
        base = wid * _SLAB

        zero16 = jnp.zeros((_LANES,), jnp.float32)

        cp_yh = pltpu.async_copy(yhat_hbm.at[pl.ds(base, _SLAB)], yh_v, sem)
        cp_yy = pltpu.async_copy(y_hbm.at[pl.ds(base, _SLAB)], yy_v, sem)
        cp_pm = pltpu.async_copy(pm_hbm.at[pl.ds(base, _SLAB)], pm_v, sem)

        @plsc.parallel_loop(0, _TAB_CHUNKS, unroll=8)
        def _(i):
            pred_t[pl.ds(i * _LANES, _LANES)] = zero16
            true_t[pl.ds(i * _LANES, _LANES)] = zero16

        cp_yh.wait()
        cp_yy.wait()
        cp_pm.wait()

        for r in range(_ROWS_PER_W):
            pred_r = pred_t.at[pl.ds(r * _NUM_PIDS, _NUM_PIDS)]
            true_r = true_t.at[pl.ds(r * _NUM_PIDS, _NUM_PIDS)]
            rb = r * _L

            @plsc.parallel_loop(0, _L // _LANES, unroll=8)
            def _(i):
                b = rb + i * _LANES
                ph = pm_v[pl.ds(b, _LANES)]
                plsc.addupdate_scatter(pred_r, [ph], yh_v[pl.ds(b, _LANES)])
                plsc.addupdate_scatter(true_r, [ph], yy_v[pl.ds(b, _LANES)])

        @plsc.parallel_loop(0, _TAB_CHUNKS, unroll=8, carry=(zero16, zero16))
        def accs(i, carry):
            a_sq, a_cnt = carry
            b = i * _LANES
            ps = pred_t[pl.ds(b, _LANES)]
            ts = true_t[pl.ds(b, _LANES)]
            valid = ts > 0.0
            diff = jnp.where(valid, ps - ts, 0.0)
            return (a_sq + diff * diff,
                    a_cnt + jnp.where(valid, 1.0, 0.0))

        acc_sq, acc_cnt = accs

        st_sq[...] = acc_sq
        st_cnt[...] = acc_cnt
        pltpu.sync_copy(st_sq, sh_sq.at[pl.ds(sid * _LANES, _LANES)])
        pltpu.sync_copy(st_cnt, sh_cnt.at[pl.ds(sid * _LANES, _LANES)])
        plsc.subcore_barrier()

        @pl.when(sid == 0)
        def _():
            pltpu.sync_copy(sh_sq, all_sq)
            pltpu.sync_copy(sh_cnt, all_cnt)

            @plsc.parallel_loop(0, _NS, unroll=4, carry=(zero16, zero16))
            def red(i, carry):
                a_sq, a_cnt = carry
                b = i * _LANES
                return (a_sq + all_sq[pl.ds(b, _LANES)],
                        a_cnt + all_cnt[pl.ds(b, _LANES)])

            v_sq, v_cnt = red
            lanes = lax.broadcasted_iota(jnp.int32, (_LANES,), 0)
            out_vec = (jnp.where(lanes == 0, jnp.sum(v_sq), 0.0)
                       + jnp.where(lanes == 1, jnp.sum(v_cnt), 0.0))
            st_sq[...] = out_vec
            pltpu.sync_copy(st_sq, out_hbm.at[cid])

    return k(yhat, y, pm)


def kernel(yhat, y, plot_mask):
    yhat = jnp.squeeze(yhat).astype(jnp.float32).reshape(_B * _L)
    y = jnp.squeeze(y).astype(jnp.float32).reshape(_B * _L)
    pm = jnp.squeeze(plot_mask).astype(jnp.int32).reshape(_B * _L)
    parts = _sc_partials(yhat, y, pm)
    total_sq = parts[0, 0] + parts[1, 0]
    total_cnt = jnp.maximum(parts[0, 1] + parts[1, 1], 1.0)
    return jnp.sqrt(total_sq / total_cnt + _EPS)


# speedup vs baseline: 1.1288x; 1.1288x over previous
import functools

import jax
import jax.numpy as jnp
from jax import lax
from jax.experimental import pallas as pl
from jax.experimental.pallas import tpu as pltpu
from jax.experimental.pallas import tpu_sc as plsc

_B, _L, _NUM_PIDS = 128, 4096, 256
_EPS = 1e-06
_LANES = 16
_NC, _NS = 2, 16
_NW = _NC * _NS
_ROWS_PER_W = _B // _NW
_ROW_CHUNKS = _L // _LANES
_TAB = _ROWS_PER_W * _NUM_PIDS
_TAB_CHUNKS = _TAB // _LANES


def _sc_partials(yhat, y, pm):
    mesh = plsc.VectorSubcoreMesh(core_axis_name="c", subcore_axis_name="s")

    @functools.partial(
        pl.kernel,
        mesh=mesh,
        out_type=jax.ShapeDtypeStruct((_NC, _LANES), jnp.float32),
        compiler_params=pltpu.CompilerParams(needs_layout_passes=False),
        scratch_types=[
            pltpu.VMEM((_ROWS_PER_W, _L), jnp.float32),
            pltpu.VMEM((_ROWS_PER_W, _L), jnp.float32),
            pltpu.VMEM((_ROWS_PER_W, _L), jnp.int32),
            pltpu.VMEM((_TAB,), jnp.float32),
            pltpu.VMEM((_TAB,), jnp.float32),
            pltpu.VMEM((_LANES,), jnp.float32),
            pltpu.VMEM((_LANES,), jnp.float32),
            pltpu.VMEM((_NS * _LANES,), jnp.float32),
            pltpu.VMEM((_NS * _LANES,), jnp.float32),
            pltpu.VMEM_SHARED((_NS * _LANES,), jnp.float32),
            pltpu.VMEM_SHARED((_NS * _LANES,), jnp.float32),
            pltpu.SemaphoreType.DMA,
        ],
    )
    def k(yhat_hbm, y_hbm, pm_hbm, out_hbm,
          yh_v, yy_v, pm_v, pred_t, true_t, st_sq, st_cnt,
          all_sq, all_cnt, sh_sq, sh_cnt, sem):
        cid = lax.axis_index("c")
        sid = lax.axis_index("s")
        wid = cid * _NS + sid
        row0 = wid * _ROWS_PER_W

        zero16 = jnp.zeros((_LANES,), jnp.float32)

        cp_yh = pltpu.async_copy(yhat_hbm.at[pl.ds(row0, _ROWS_PER_W)],
                                 yh_v, sem)
        cp_yy = pltpu.async_copy(y_hbm.at[pl.ds(row0, _ROWS_PER_W)],
                                 yy_v, sem)
        cp_pm = pltpu.async_copy(pm_hbm.at[pl.ds(row0, _ROWS_PER_W)],
                                 pm_v, sem)

        @plsc.parallel_loop(0, _TAB_CHUNKS, unroll=8)
        def _(i):
            pred_t[pl.ds(i * _LANES, _LANES)] = zero16
            true_t[pl.ds(i * _LANES, _LANES)] = zero16

        cp_yh.wait()
        cp_yy.wait()
        cp_pm.wait()

        for r in range(_ROWS_PER_W):
            pred_r = pred_t.at[pl.ds(r * _NUM_PIDS, _NUM_PIDS)]
            true_r = true_t.at[pl.ds(r * _NUM_PIDS, _NUM_PIDS)]

            @plsc.parallel_loop(0, _ROW_CHUNKS, unroll=8)
            def _(i):
                b = i * _LANES
                ph = pm_v[r, pl.ds(b, _LANES)]
                plsc.addupdate_scatter(pred_r, [ph], yh_v[r, pl.ds(b, _LANES)])
                plsc.addupdate_scatter(true_r, [ph], yy_v[r, pl.ds(b, _LANES)])

        @plsc.parallel_loop(0, _TAB_CHUNKS, unroll=8, carry=(zero16, zero16))
        def accs(i, carry):
            a_sq, a_cnt = carry
            b = i * _LANES
            ps = pred_t[pl.ds(b, _LANES)]
            ts = true_t[pl.ds(b, _LANES)]
            valid = ts > 0.0
            diff = jnp.where(valid, ps - ts, 0.0)
            return (a_sq + diff * diff,
                    a_cnt + jnp.where(valid, 1.0, 0.0))

        acc_sq, acc_cnt = accs

        st_sq[...] = acc_sq
        st_cnt[...] = acc_cnt
        pltpu.sync_copy(st_sq, sh_sq.at[pl.ds(sid * _LANES, _LANES)])
        pltpu.sync_copy(st_cnt, sh_cnt.at[pl.ds(sid * _LANES, _LANES)])
        plsc.subcore_barrier()

        @pl.when(sid == 0)
        def _():
            pltpu.sync_copy(sh_sq, all_sq)
            pltpu.sync_copy(sh_cnt, all_cnt)

            @plsc.parallel_loop(0, _NS, unroll=4, carry=(zero16, zero16))
            def red(i, carry):
                a_sq, a_cnt = carry
                b = i * _LANES
                return (a_sq + all_sq[pl.ds(b, _LANES)],
                        a_cnt + all_cnt[pl.ds(b, _LANES)])

            v_sq, v_cnt = red
            lanes = lax.broadcasted_iota(jnp.int32, (_LANES,), 0)
            out_vec = (jnp.where(lanes == 0, jnp.sum(v_sq), 0.0)
                       + jnp.where(lanes == 1, jnp.sum(v_cnt), 0.0))
            st_sq[...] = out_vec
            pltpu.sync_copy(st_sq, out_hbm.at[cid])

    return k(yhat, y, pm)


def kernel(yhat, y, plot_mask):
    yhat = jnp.squeeze(yhat).astype(jnp.float32)
    y = jnp.squeeze(y).astype(jnp.float32)
    pm = jnp.squeeze(plot_mask).astype(jnp.int32)
    parts = _sc_partials(yhat, y, pm)
    total_sq = parts[0, 0] + parts[1, 0]
    total_cnt = jnp.maximum(parts[0, 1] + parts[1, 1], 1.0)
    return jnp.sqrt(total_sq / total_cnt + _EPS)
